# baseline (speedup 1.0000x reference)
import functools
import math

import jax
import jax.numpy as jnp
from jax import lax
from jax.experimental import pallas as pl
from jax.experimental.pallas import tpu as pltpu
from jax.experimental.pallas import tpu_sc as plsc

_VOCAB = 100000
_EMBED = 64
_BATCH = 16384
_LMIN = math.log(0.1)
_LMAX = math.log(10.0)
_MUC = math.sqrt(2.0)

_NC = 2
_NS = 16
_NW = _NC * _NS
_BPW = _BATCH // _NW
_GB = 8
_NG = _BPW // _GB

_mesh = plsc.VectorSubcoreMesh(core_axis_name="c", subcore_axis_name="s")


def _row_buf():
    return pltpu.VMEM((_GB, 8, _EMBED), jnp.float32)


@functools.partial(
    pl.kernel,
    out_type=jax.ShapeDtypeStruct((_NW * 16,), jnp.float32),
    mesh=_mesh,
    scratch_types=[
        pltpu.VMEM((_BPW + 16,), jnp.int32),
        pltpu.VMEM((_BPW + 16,), jnp.int32),
        pltpu.VMEM((_BPW + 16,), jnp.int32),
        [_row_buf() for _ in range(6)],
        [_row_buf() for _ in range(6)],
        pltpu.VMEM((16,), jnp.float32),
        pltpu.SemaphoreType.DMA,
        pltpu.SemaphoreType.DMA,
    ],
)
def _gauss_loss_sc(wi_hbm, wj_hbm, wn_hbm, mu_hbm, mup_hbm, mun_hbm,
                   ls_hbm, lsp_hbm, lsn_hbm, out_hbm,
                   ii_v, ij_v, in_v, bufs0, bufs1,
                   acc_v, sem0, sem1):
    cidx = lax.axis_index("c")
    sidx = lax.axis_index("s")
    wid = sidx * _NC + cidx
    base = wid * _BPW

    tables = (mu_hbm, mup_hbm, mun_hbm, ls_hbm, lsp_hbm, lsn_hbm)
    lane = lax.iota(jnp.int32, 16)
    zeros16 = jnp.zeros((16,), jnp.float32)

    def clip_mu(x):
        return jnp.minimum(jnp.maximum(x, -_MUC), _MUC)

    def clip_ls(x):
        return jnp.minimum(jnp.maximum(x, _LMIN), _LMAX)

    pltpu.sync_copy(wi_hbm.at[pl.ds(base, _BPW)], ii_v.at[pl.ds(0, _BPW)])
    pltpu.sync_copy(wj_hbm.at[pl.ds(base, _BPW)], ij_v.at[pl.ds(0, _BPW)])
    pltpu.sync_copy(wn_hbm.at[pl.ds(base, _BPW)], in_v.at[pl.ds(0, _BPW)])

    def fire(off16, hi, bufs, sem):
        iv = ii_v[pl.ds(off16, 16)]
        jv = ij_v[pl.ds(off16, 16)]
        nv = in_v[pl.ds(off16, 16)]
        idx_by_table = (iv, jv, nv, iv, jv, nv)
        cps = []
        for u in range(_GB):
            for t in range(6):
                idx = idx_by_table[t][hi * 8 + u]
                al = pl.multiple_of((idx // 8) * 8, 8)
                cps.append(pltpu.async_copy(
                    tables[t].at[pl.ds(al, 8)],
                    bufs[t].at[u],
                    sem,
                ))
        return cps

    def compute(off16, hi, bufs, acc):
        rmi, rmj, rmn, rsi, rsj, rsn = bufs
        iv = ii_v[pl.ds(off16, 16)]
        jv = ij_v[pl.ds(off16, 16)]
        nv = in_v[pl.ds(off16, 16)]
        for u in range(_GB):
            ri = lax.rem(iv[hi * 8 + u], 8)
            rj = lax.rem(jv[hi * 8 + u], 8)
            rn = lax.rem(nv[hi * 8 + u], 8)
            vsum = zeros16
            for k in range(_EMBED // 16):
                sl = pl.ds(k * 16, 16)
                mi = clip_mu(rmi[u, ri, sl])
                mj = clip_mu(rmj[u, rj, sl])
                mn = clip_mu(rmn[u, rn, sl])
                li = clip_ls(rsi[u, ri, sl])
                lj = clip_ls(rsj[u, rj, sl])
                ln = clip_ls(rsn[u, rn, sl])
                inv_si = jnp.exp(-li)
                dj = mi - mj
                dn = mi - mn
                num = jnp.exp(lj) - jnp.exp(ln) + dj * dj - dn * dn
                vsum = vsum + num * inv_si + lj - ln
            s = [vsum[i] for i in range(16)]
            while len(s) > 1:
                s = [s[2 * i] + s[2 * i + 1] for i in range(len(s) // 2)]
            acc = acc + jnp.maximum(1.0 + 0.5 * s[0], 0.0)
        return acc

    def grp_body(g, acc):
        parity = lax.rem(g, 2)

        def even(acc):
            for cp in fire(g * _GB, 0, bufs0, sem0):
                cp.wait()
            return compute(g * _GB, 0, bufs0, acc)

        def odd(acc):
            for cp in fire((g - 1) * _GB, 1, bufs1, sem1):
                cp.wait()
            return compute((g - 1) * _GB, 1, bufs1, acc)

        return lax.cond(parity == 0, even, odd, acc)

    acc = lax.fori_loop(0, _NG, grp_body, jnp.float32(0.0))

    acc_v[...] = jnp.where(lane == 0, acc, 0.0)
    pltpu.sync_copy(acc_v, out_hbm.at[pl.ds(wid * 16, 16)])


def kernel(words_i, words_j, words_n, mu, mu_pos, mu_neg,
           log_sigma, log_sigma_pos, log_sigma_neg):
    partials = _gauss_loss_sc(
        words_i.astype(jnp.int32), words_j.astype(jnp.int32),
        words_n.astype(jnp.int32), mu, mu_pos, mu_neg,
        log_sigma, log_sigma_pos, log_sigma_neg)
    return jnp.sum(partials) * (1.0 / _BATCH)


# speedup vs baseline: 1.2439x; 1.2439x over previous
import functools
import math

import jax
import jax.numpy as jnp
from jax import lax
from jax.experimental import pallas as pl
from jax.experimental.pallas import tpu as pltpu
from jax.experimental.pallas import tpu_sc as plsc

_VOCAB = 100000
_EMBED = 64
_BATCH = 16384
_LMIN = math.log(0.1)
_LMAX = math.log(10.0)
_MUC = math.sqrt(2.0)

_NC = 2
_NS = 16
_NW = _NC * _NS
_BPW = _BATCH // _NW
_GB = 8
_NG = _BPW // _GB

_mesh = plsc.VectorSubcoreMesh(core_axis_name="c", subcore_axis_name="s")


def _row_buf():
    return pltpu.VMEM((_GB, 8, 2 * _EMBED), jnp.float32)


@functools.partial(
    pl.kernel,
    out_type=jax.ShapeDtypeStruct((_NW * 16,), jnp.float32),
    mesh=_mesh,
    scratch_types=[
        pltpu.VMEM((_BPW + 16,), jnp.int32),
        pltpu.VMEM((_BPW + 16,), jnp.int32),
        pltpu.VMEM((_BPW + 16,), jnp.int32),
        [_row_buf() for _ in range(3)],
        [_row_buf() for _ in range(3)],
        pltpu.VMEM((16,), jnp.float32),
        pltpu.SemaphoreType.DMA,
        pltpu.SemaphoreType.DMA,
    ],
)
def _gauss_loss_sc(wi_hbm, wj_hbm, wn_hbm, pi_hbm, pj_hbm, pn_hbm, out_hbm,
                   ii_v, ij_v, in_v, bufs0, bufs1, acc_v, sem0, sem1):
    cidx = lax.axis_index("c")
    sidx = lax.axis_index("s")
    wid = sidx * _NC + cidx
    base = wid * _BPW

    tables = (pi_hbm, pj_hbm, pn_hbm)
    lane = lax.iota(jnp.int32, 16)
    zeros16 = jnp.zeros((16,), jnp.float32)

    def clip_mu(x):
        return jnp.minimum(jnp.maximum(x, -_MUC), _MUC)

    def clip_ls(x):
        return jnp.minimum(jnp.maximum(x, _LMIN), _LMAX)

    pltpu.sync_copy(wi_hbm.at[pl.ds(base, _BPW)], ii_v.at[pl.ds(0, _BPW)])
    pltpu.sync_copy(wj_hbm.at[pl.ds(base, _BPW)], ij_v.at[pl.ds(0, _BPW)])
    pltpu.sync_copy(wn_hbm.at[pl.ds(base, _BPW)], in_v.at[pl.ds(0, _BPW)])

    def fire(off16, hi, bufs, sem):
        iv = ii_v[pl.ds(off16, 16)]
        jv = ij_v[pl.ds(off16, 16)]
        nv = in_v[pl.ds(off16, 16)]
        idx_by_table = (iv, jv, nv)
        for u in range(_GB):
            for t in range(3):
                idx = idx_by_table[t][hi * 8 + u]
                al = pl.multiple_of((idx // 8) * 8, 8)
                pltpu.async_copy(
                    tables[t].at[pl.ds(al, 8)],
                    bufs[t].at[u],
                    sem,
                )

    def drain(bufs, sem):
        for t in range(3):
            for u in range(_GB):
                pltpu.make_async_copy(
                    tables[t].at[pl.ds(0, 8)], bufs[t].at[u], sem
                ).wait()

    def compute(off16, hi, bufs, acc):
        rpi, rpj, rpn = bufs
        iv = ii_v[pl.ds(off16, 16)]
        jv = ij_v[pl.ds(off16, 16)]
        nv = in_v[pl.ds(off16, 16)]
        for u in range(_GB):
            ri = lax.rem(iv[hi * 8 + u], 8)
            rj = lax.rem(jv[hi * 8 + u], 8)
            rn = lax.rem(nv[hi * 8 + u], 8)
            vsum = zeros16
            for k in range(_EMBED // 16):
                sl = pl.ds(k * 16, 16)
                sh = pl.ds(_EMBED + k * 16, 16)
                mi = clip_mu(rpi[u, ri, sl])
                mj = clip_mu(rpj[u, rj, sl])
                mn = clip_mu(rpn[u, rn, sl])
                li = clip_ls(rpi[u, ri, sh])
                lj = clip_ls(rpj[u, rj, sh])
                ln = clip_ls(rpn[u, rn, sh])
                inv_si = jnp.exp(-li)
                dj = mi - mj
                dn = mi - mn
                num = jnp.exp(lj) - jnp.exp(ln) + dj * dj - dn * dn
                vsum = vsum + num * inv_si + lj - ln
            s = [vsum[i] for i in range(16)]
            while len(s) > 1:
                s = [s[2 * i] + s[2 * i + 1] for i in range(len(s) // 2)]
            acc = acc + jnp.maximum(1.0 + 0.5 * s[0], 0.0)
        return acc

    fire(0, 0, bufs0, sem0)

    def grp_body(g, acc):
        parity = lax.rem(g, 2)

        @pl.when(jnp.logical_and(parity == 0, g + 1 < _NG))
        def _():
            fire(g * _GB, 1, bufs1, sem1)

        @pl.when(jnp.logical_and(parity == 1, g + 1 < _NG))
        def _():
            fire((g + 1) * _GB, 0, bufs0, sem0)

        def even(acc):
            drain(bufs0, sem0)
            return compute(g * _GB, 0, bufs0, acc)

        def odd(acc):
            drain(bufs1, sem1)
            return compute((g - 1) * _GB, 1, bufs1, acc)

        return lax.cond(parity == 0, even, odd, acc)

    acc = lax.fori_loop(0, _NG, grp_body, jnp.float32(0.0))

    acc_v[...] = jnp.where(lane == 0, acc, 0.0)
    pltpu.sync_copy(acc_v, out_hbm.at[pl.ds(wid * 16, 16)])


def kernel(words_i, words_j, words_n, mu, mu_pos, mu_neg,
           log_sigma, log_sigma_pos, log_sigma_neg):
    pair_i = jnp.concatenate([mu, log_sigma], axis=1)
    pair_j = jnp.concatenate([mu_pos, log_sigma_pos], axis=1)
    pair_n = jnp.concatenate([mu_neg, log_sigma_neg], axis=1)
    partials = _gauss_loss_sc(
        words_i.astype(jnp.int32), words_j.astype(jnp.int32),
        words_n.astype(jnp.int32), pair_i, pair_j, pair_n)
    return jnp.sum(partials) * (1.0 / _BATCH)
